# speedup vs baseline: 95.6064x; 95.6064x over previous
import jax
import jax.numpy as jnp
from jax.experimental import pallas as pl
from jax.experimental.pallas import tpu as pltpu

MASK_TOKEN_ID = 0
K = 3
EPS = 1e-6
ROWS_PER_BLOCK = 8


def _head_kernel(ids_ref, prm_ref, x_ref, o_ref):
    x = x_ref[:]
    v = x.shape[1]

    m = jnp.max(x, axis=1, keepdims=True)
    e = jnp.exp(x - m)
    z = jnp.sum(e, axis=1, keepdims=True)
    p = e / z
    neg_ent = jnp.sum(p * jnp.log(p + 1e-10), axis=1, keepdims=True)

    iota = jax.lax.broadcasted_iota(jnp.int32, x.shape, 1)
    xm = x
    tvals = []
    tidx = []
    for _ in range(K):
        vk = jnp.max(xm, axis=1, keepdims=True)
        ik = jnp.min(jnp.where(xm == vk, iota, v), axis=1, keepdims=True)
        tvals.append(vk)
        tidx.append(ik)
        xm = jnp.where(iota == ik, -jnp.inf, xm)

    te = [jnp.exp(t - tvals[0]) for t in tvals]
    tz = te[0] + te[1] + te[2]

    raw_scale = prm_ref[0, 0]
    raw_centre_neg = prm_ref[0, 1]
    raw_steep = prm_ref[0, 2]
    scale = jax.nn.sigmoid(raw_scale)
    centre = -jax.nn.softplus(raw_centre_neg) - EPS
    steep = jax.nn.softplus(raw_steep) + EPS

    lam = scale * jax.nn.sigmoid(steep * (neg_ent - centre))
    ids = ids_ref[:]
    lam = jnp.where(ids == MASK_TOKEN_ID, lam, 0.0)

    acc = jnp.where(iota == ids, 1.0 - lam, 0.0)
    for k in range(K):
        acc = acc + jnp.where(iota == tidx[k], lam * (te[k] / tz), 0.0)
    o_ref[:] = acc


def kernel(input_ids, logits_prelim, raw_scale, raw_centre_neg, raw_steep, raw_temperature):
    b, s, v = logits_prelim.shape
    n = b * s
    r = ROWS_PER_BLOCK
    x = logits_prelim.reshape(n, v)
    ids = input_ids.reshape(n, 1).astype(jnp.int32)
    prm = jnp.stack(
        [raw_scale, raw_centre_neg, raw_steep, raw_temperature]
    ).reshape(1, 4).astype(jnp.float32)

    out = pl.pallas_call(
        _head_kernel,
        grid=(n // r,),
        in_specs=[
            pl.BlockSpec((r, 1), lambda i: (i, 0)),
            pl.BlockSpec(memory_space=pltpu.SMEM),
            pl.BlockSpec((r, v), lambda i: (i, 0)),
        ],
        out_specs=pl.BlockSpec((r, v), lambda i: (i, 0)),
        out_shape=jax.ShapeDtypeStruct((n, v), jnp.float32),
        compiler_params=pltpu.CompilerParams(
            dimension_semantics=("arbitrary",),
        ),
    )(ids, prm, x)
    return out.reshape(b, s, v)
